# Initial kernel scaffold; baseline (speedup 1.0000x reference)
import jax
import jax.numpy as jnp
from jax.experimental import pallas as pl


def kernel(distances):
    raise NotImplementedError("write your pallas kernel here")


# speedup vs baseline: 13.3129x; 13.3129x over previous
import jax
import jax.numpy as jnp
from jax.experimental import pallas as pl

_K = 20
_BITS = 31


def _body(x_ref, o_ref):
    x = x_ref[...]
    xi = jax.lax.bitcast_convert_type(x, jnp.int32)
    rows = x.shape[0]

    prefix = jnp.zeros((rows, 1), jnp.int32)
    k_rem = jnp.full((rows, 1), _K, jnp.int32)

    for i in range(_BITS):
        b = _BITS - 1 - i
        cand = prefix * 2
        c0 = jnp.sum((xi >> b) == cand, axis=1, keepdims=True, dtype=jnp.int32)
        go1 = k_rem >= c0
        prefix = cand + go1.astype(jnp.int32)
        k_rem = k_rem - jnp.where(go1, c0, 0)

    t = jax.lax.bitcast_convert_type(prefix, jnp.float32)

    less = x < t
    c_less = jnp.sum(less, axis=1, keepdims=True, dtype=jnp.int32)
    s_less = jnp.sum(jnp.where(less, x, 0.0), axis=1, keepdims=True)
    sum_k = s_less + (jnp.float32(_K) - c_less.astype(jnp.float32)) * t

    tk = t + jnp.float32(1e-10)
    inv = 1.0 / (jnp.float32(_K) * tk - sum_k)
    o_ref[...] = jnp.maximum((tk - x) * inv, 0.0)


def kernel(distances):
    n_rows, n_cols = distances.shape
    block_rows = 256 if n_rows % 256 == 0 else n_rows
    grid = (n_rows // block_rows,)
    return pl.pallas_call(
        _body,
        grid=grid,
        in_specs=[pl.BlockSpec((block_rows, n_cols), lambda i: (i, 0))],
        out_specs=pl.BlockSpec((block_rows, n_cols), lambda i: (i, 0)),
        out_shape=jax.ShapeDtypeStruct((n_rows, n_cols), jnp.float32),
    )(distances)


# speedup vs baseline: 20.8784x; 1.5683x over previous
import jax
import jax.numpy as jnp
from jax.experimental import pallas as pl

_K = 20
_BITS = 31


def _body(x_ref, o_ref):
    x = x_ref[...]
    rows = x.shape[0]

    lo = jnp.zeros((rows, 1), jnp.int32)
    for b in range(_BITS - 1, -1, -1):
        mid_bits = lo + (1 << b)
        mid = jax.lax.bitcast_convert_type(mid_bits, jnp.float32)
        n_mid = jnp.sum(x < mid, axis=1, keepdims=True, dtype=jnp.int32)
        lo = jnp.where(n_mid <= _K, mid_bits, lo)

    t = jax.lax.bitcast_convert_type(lo, jnp.float32)

    less = x < t
    c_less = jnp.sum(less, axis=1, keepdims=True, dtype=jnp.int32)
    s_less = jnp.sum(jnp.where(less, x, 0.0), axis=1, keepdims=True)
    sum_k = s_less + (jnp.float32(_K) - c_less.astype(jnp.float32)) * t

    tk = t + jnp.float32(1e-10)
    inv = 1.0 / (jnp.float32(_K) * tk - sum_k)
    o_ref[...] = jnp.maximum((tk - x) * inv, 0.0)


def kernel(distances):
    n_rows, n_cols = distances.shape
    block_rows = 256 if n_rows % 256 == 0 else n_rows
    grid = (n_rows // block_rows,)
    return pl.pallas_call(
        _body,
        grid=grid,
        in_specs=[pl.BlockSpec((block_rows, n_cols), lambda i: (i, 0))],
        out_specs=pl.BlockSpec((block_rows, n_cols), lambda i: (i, 0)),
        out_shape=jax.ShapeDtypeStruct((n_rows, n_cols), jnp.float32),
    )(distances)
